# baseline (speedup 1.0000x reference)
import functools
import math

import jax
import jax.numpy as jnp
from jax import lax
from jax.experimental import pallas as pl
from jax.experimental.pallas import tpu as pltpu
from jax.experimental.pallas import tpu_sc as plsc

EMB = 64
PAIR = 128
SCALE = math.sqrt(EMB)
NC = 2
NS = 16
NW = NC * NS
CHUNK = 128
LANES = 16
GRPS = CHUNK // LANES
NBUF = 4
PREP_BLK = 4096
HALF_V = 4096 * 123


def _tc_prep(table_t):
    def body(lo_ref, hi_ref, out_ref):
        eye = jnp.eye(EMB, dtype=jnp.float32)
        dn = (((0,), (0,)), ((), ()))
        lo = lax.dot_general(lo_ref[:, :], eye, dn,
                             preferred_element_type=jnp.float32)
        hi = lax.dot_general(hi_ref[:, :], eye, dn,
                             preferred_element_type=jnp.float32)
        out_ref[:, :] = jnp.concatenate([lo, hi], axis=1)

    nblk = HALF_V // PREP_BLK
    last_blk = (table_t.shape[1] - 1) // PREP_BLK
    return pl.pallas_call(
        body,
        grid=(nblk,),
        in_specs=[
            pl.BlockSpec((EMB, PREP_BLK), lambda i: (0, i)),
            pl.BlockSpec(
                (EMB, PREP_BLK), lambda i: (0, jnp.minimum(i + nblk, last_blk))
            ),
        ],
        out_specs=pl.BlockSpec((PREP_BLK, PAIR), lambda i: (i, 0)),
        out_shape=jax.ShapeDtypeStruct((HALF_V, PAIR), jnp.float32),
    )(table_t, table_t)


def _sc_embed(tokens_t, table_r):
    C, R = tokens_t.shape
    n_chunks = C
    mesh = plsc.VectorSubcoreMesh(core_axis_name="c", subcore_axis_name="s")

    @functools.partial(
        pl.kernel,
        out_type=jax.ShapeDtypeStruct((C, EMB, R), jnp.float32),
        mesh=mesh,
        scratch_types=[
            pltpu.VMEM((n_chunks, CHUNK), jnp.int32),
            pltpu.VMEM((NBUF, CHUNK), jnp.int32),
            pltpu.VMEM((NBUF, CHUNK, PAIR), jnp.float32),
            pltpu.VMEM((NBUF, EMB, CHUNK), jnp.float32),
            pltpu.SemaphoreType.DMA((NBUF, 2)),
            pltpu.SemaphoreType.DMA((NBUF,)),
        ],
        compiler_params=pltpu.CompilerParams(
            use_tc_tiling_on_sc=True, needs_layout_passes=False
        ),
    )
    def body(tokens_hbm, table_hbm, out_hbm, idx_v, idx2, graw, sbuf,
             gsem, ssem):
        wid = lax.axis_index("s") * NC + lax.axis_index("c")
        col0 = wid * CHUNK
        pltpu.sync_copy(tokens_hbm.at[:, pl.ds(col0, CHUNK)], idx_v)

        def issue_gather(jj, b):
            for g in range(GRPS):
                sl = pl.ds(g * LANES, LANES)
                v = idx_v[jj, sl]
                hi = (v >= HALF_V).astype(jnp.int32)
                idx2[b, sl] = v - hi * HALF_V
            for h in range(2):
                hs = pl.ds(h * (CHUNK // 2), CHUNK // 2)
                pltpu.async_copy(
                    table_hbm.at[idx2.at[b, hs]], graw.at[b, hs], gsem.at[b, h]
                )

        for b in range(NBUF):
            issue_gather(b, b)

        def outer(gi, carry):
            for b in range(NBUF):
                j = gi * NBUF + b
                for h in range(2):
                    hs = pl.ds(h * (CHUNK // 2), CHUNK // 2)
                    pltpu.make_async_copy(
                        table_hbm.at[idx2.at[b, hs]], graw.at[b, hs],
                        gsem.at[b, h],
                    ).wait()
                @pl.when(j >= NBUF)
                def _():
                    pltpu.make_async_copy(
                        sbuf.at[b],
                        out_hbm.at[0, :, pl.ds(col0, CHUNK)],
                        ssem.at[b],
                    ).wait()

                for grp in range(GRPS):
                    rows = lax.iota(jnp.int32, LANES) + grp * LANES
                    sl = pl.ds(grp * LANES, LANES)
                    par = (idx_v[j, sl] >= HALF_V).astype(jnp.int32) << 6

                    @plsc.parallel_loop(0, EMB, unroll=8)
                    def _rp(e):
                        cols = par + e
                        v = plsc.load_gather(graw.at[b], [rows, cols])
                        sbuf[b, e, sl] = v * SCALE

                pltpu.async_copy(
                    sbuf.at[b],
                    out_hbm.at[j, :, pl.ds(col0, CHUNK)],
                    ssem.at[b],
                )

                @pl.when(j + NBUF < n_chunks)
                def _():
                    issue_gather(j + NBUF, b)
            return carry

        lax.fori_loop(0, n_chunks // NBUF, outer, 0)

        for b in range(NBUF):
            pltpu.make_async_copy(
                sbuf.at[b], out_hbm.at[0, :, pl.ds(col0, CHUNK)], ssem.at[b]
            ).wait()

    return body(tokens_t, table_r)


def kernel(tokens, table):
    table_r = _tc_prep(table.T)
    out_t = _sc_embed(tokens.T, table_r)
    return jnp.transpose(out_t, (2, 0, 1))


# speedup vs baseline: 1.0335x; 1.0335x over previous
import functools
import math

import jax
import jax.numpy as jnp
from jax import lax
from jax.experimental import pallas as pl
from jax.experimental.pallas import tpu as pltpu
from jax.experimental.pallas import tpu_sc as plsc

EMB = 64
PAIR = 128
SCALE = math.sqrt(EMB)
NC = 2
NS = 16
NW = NC * NS
CHUNK = 128
LANES = 16
GRPS = CHUNK // LANES
NBUF = 4
PREP_BLK = 8192
HALF_V = 8192 * 62


def _tc_prep(table_t):
    def body(lo_ref, hi_ref, out_ref):
        eye = jnp.eye(EMB, dtype=jnp.float32)
        dn = (((0,), (0,)), ((), ()))
        lo = lax.dot_general(lo_ref[:, :], eye, dn,
                             preferred_element_type=jnp.float32)
        hi = lax.dot_general(hi_ref[:, :], eye, dn,
                             preferred_element_type=jnp.float32)
        out_ref[:, :] = jnp.concatenate([lo, hi], axis=1)

    nblk = HALF_V // PREP_BLK
    last_blk = (table_t.shape[1] - 1) // PREP_BLK
    return pl.pallas_call(
        body,
        grid=(nblk,),
        in_specs=[
            pl.BlockSpec((EMB, PREP_BLK), lambda i: (0, i)),
            pl.BlockSpec(
                (EMB, PREP_BLK), lambda i: (0, jnp.minimum(i + nblk, last_blk))
            ),
        ],
        out_specs=pl.BlockSpec((PREP_BLK, PAIR), lambda i: (i, 0)),
        out_shape=jax.ShapeDtypeStruct((HALF_V, PAIR), jnp.float32),
    )(table_t, table_t)


def _sc_embed(tokens_t, table_r):
    C, R = tokens_t.shape
    n_chunks = C
    mesh = plsc.VectorSubcoreMesh(core_axis_name="c", subcore_axis_name="s")

    @functools.partial(
        pl.kernel,
        out_type=jax.ShapeDtypeStruct((C, EMB, R), jnp.float32),
        mesh=mesh,
        scratch_types=[
            pltpu.VMEM((n_chunks, CHUNK), jnp.int32),
            pltpu.VMEM((NBUF, CHUNK), jnp.int32),
            pltpu.VMEM((NBUF, CHUNK, PAIR), jnp.float32),
            pltpu.VMEM((NBUF, EMB, CHUNK), jnp.float32),
            pltpu.SemaphoreType.DMA((NBUF, 2)),
            pltpu.SemaphoreType.DMA((NBUF,)),
        ],
        compiler_params=pltpu.CompilerParams(
            use_tc_tiling_on_sc=True, needs_layout_passes=False
        ),
    )
    def body(tokens_hbm, table_hbm, out_hbm, idx_v, idx2, graw, sbuf,
             gsem, ssem):
        wid = lax.axis_index("s") * NC + lax.axis_index("c")
        col0 = wid * CHUNK
        pltpu.sync_copy(tokens_hbm.at[:, pl.ds(col0, CHUNK)], idx_v)

        def issue_gather(jj, b):
            for g in range(GRPS):
                sl = pl.ds(g * LANES, LANES)
                v = idx_v[jj, sl]
                hi = (v >= HALF_V).astype(jnp.int32)
                idx2[b, sl] = v - hi * HALF_V
            for h in range(2):
                hs = pl.ds(h * (CHUNK // 2), CHUNK // 2)
                pltpu.async_copy(
                    table_hbm.at[idx2.at[b, hs]], graw.at[b, hs], gsem.at[b, h]
                )

        for b in range(NBUF):
            issue_gather(b, b)

        def outer(gi, carry):
            for b in range(NBUF):
                j = gi * NBUF + b
                for h in range(2):
                    hs = pl.ds(h * (CHUNK // 2), CHUNK // 2)
                    pltpu.make_async_copy(
                        table_hbm.at[idx2.at[b, hs]], graw.at[b, hs],
                        gsem.at[b, h],
                    ).wait()
                @pl.when(j >= NBUF)
                def _():
                    pltpu.make_async_copy(
                        sbuf.at[b],
                        out_hbm.at[0, :, pl.ds(col0, CHUNK)],
                        ssem.at[b],
                    ).wait()

                for grp in range(GRPS):
                    rows = lax.iota(jnp.int32, LANES) + grp * LANES
                    sl = pl.ds(grp * LANES, LANES)
                    par = (idx_v[j, sl] >= HALF_V).astype(jnp.int32) << 6

                    @plsc.parallel_loop(0, EMB, unroll=16)
                    def _rp(e):
                        cols = par + e
                        v = plsc.load_gather(graw.at[b], [rows, cols])
                        sbuf[b, e, sl] = v * SCALE

                pltpu.async_copy(
                    sbuf.at[b],
                    out_hbm.at[j, :, pl.ds(col0, CHUNK)],
                    ssem.at[b],
                )

                @pl.when(j + NBUF < n_chunks)
                def _():
                    issue_gather(j + NBUF, b)
            return carry

        lax.fori_loop(0, n_chunks // NBUF, outer, 0)

        for b in range(NBUF):
            pltpu.make_async_copy(
                sbuf.at[b], out_hbm.at[0, :, pl.ds(col0, CHUNK)], ssem.at[b]
            ).wait()

    return body(tokens_t, table_r)


def kernel(tokens, table):
    table_r = _tc_prep(table.T)
    out_t = _sc_embed(tokens.T, table_r)
    return jnp.transpose(out_t, (2, 0, 1))
